# baseline (speedup 1.0000x reference)
import functools

import jax
import jax.numpy as jnp
from jax import lax
from jax.experimental import pallas as pl
from jax.experimental.pallas import tpu as pltpu
from jax.experimental.pallas import tpu_sc as plsc

B = 16384
D = 32
ENT = 1000000
NC = 2
NS = 16
NW = NC * NS
RPW = B // NW
NCHUNK = 4
CHUNK = RPW // NCHUNK

TCB = 16000
TCT = 8000
TCG = 63


def _tc_body(x_hbm, o_hbm, vin, vout, vtin, vtout, sin, sout, stail):
    g = pl.program_id(0)
    par = g % 2

    def in_copy(gi, p):
        return pltpu.make_async_copy(
            x_hbm.at[:, pl.ds(gi * TCB, TCB)], vin.at[p], sin.at[p])

    def out_copy(gi, p):
        return pltpu.make_async_copy(
            vout.at[p], o_hbm.at[pl.ds(gi * TCB, TCB), :], sout.at[p])

    tail_in = pltpu.make_async_copy(
        x_hbm.at[:, pl.ds((TCG - 1) * TCB, TCT)], vtin, stail)
    tail_out = pltpu.make_async_copy(
        vtout, o_hbm.at[pl.ds((TCG - 1) * TCB, TCT), :], stail)

    @pl.when(g == 0)
    def _():
        in_copy(0, 0).start()
        in_copy(1, 1).start()

    @pl.when(g == TCG - 2)
    def _():
        tail_in.start()

    @pl.when(jnp.logical_and(g + 1 < TCG - 1, g > 0))
    def _():
        in_copy(g + 1, (g + 1) % 2).start()

    @pl.when(g >= 2)
    def _():
        out_copy(g - 2, par).wait()

    @pl.when(g < TCG - 1)
    def _():
        in_copy(g, par).wait()
        vout[par, :, :] = vin[par].T

    @pl.when(g == TCG - 1)
    def _():
        tail_in.wait()
        vtout[...] = vtin[...].T

    @pl.when(g < TCG - 1)
    def _():
        out_copy(g, par).start()

    @pl.when(g == TCG - 1)
    def _():
        tail_out.start()
        out_copy(g - 1, 1 - par).wait()
        tail_out.wait()


_tc_pack = pl.pallas_call(
    _tc_body,
    grid=(TCG,),
    in_specs=[pl.BlockSpec(memory_space=pltpu.MemorySpace.HBM)],
    out_specs=pl.BlockSpec(memory_space=pltpu.MemorySpace.HBM),
    out_shape=jax.ShapeDtypeStruct((ENT, D), jnp.float32),
    scratch_shapes=[
        pltpu.VMEM((2, D, TCB), jnp.float32),
        pltpu.VMEM((2, TCB, D), jnp.float32),
        pltpu.VMEM((D, TCT), jnp.float32),
        pltpu.VMEM((TCT, D), jnp.float32),
        pltpu.SemaphoreType.DMA((2,)),
        pltpu.SemaphoreType.DMA((2,)),
        pltpu.SemaphoreType.DMA,
    ],
)


def _rsqrt(x):
    i = lax.bitcast_convert_type(x, jnp.int32)
    y = lax.bitcast_convert_type(jnp.int32(0x5F3759DF) - (i >> 1),
                                 jnp.float32)
    for _ in range(3):
        y = y * (1.5 - 0.5 * x * y * y)
    return y


@functools.partial(
    pl.kernel,
    mesh=plsc.VectorSubcoreMesh(core_axis_name="c", subcore_axis_name="s"),
    compiler_params=pltpu.CompilerParams(
        needs_layout_passes=False, use_tc_tiling_on_sc=False),
    out_type=(
        jax.ShapeDtypeStruct((B, D), jnp.float32),
        jax.ShapeDtypeStruct((B, D), jnp.float32),
        jax.ShapeDtypeStruct((B, D), jnp.float32),
    ),
    scratch_types=[
        pltpu.VMEM((NCHUNK, CHUNK), jnp.int32),
        pltpu.VMEM((NCHUNK, CHUNK), jnp.int32),
        pltpu.VMEM((NCHUNK, CHUNK), jnp.int32),
        pltpu.VMEM((RPW, D), jnp.float32),
        pltpu.VMEM((RPW, D), jnp.float32),
        pltpu.VMEM((RPW, D), jnp.float32),
        pltpu.VMEM((RPW, D), jnp.float32),
        pltpu.VMEM((RPW, D), jnp.float32),
        pltpu.VMEM((RPW, D), jnp.float32),
        pltpu.SemaphoreType.DMA,
    ],
)
def _transd_sc(h3, r3, t3, eE, rE, eEP, rEP, hout, reout, tout,
               hv, rv, tv, hp, he, tp, te, rp, reb, sem):
    wid = lax.axis_index("s") * NC + lax.axis_index("c")
    pltpu.sync_copy(h3.at[wid], hv)
    pltpu.sync_copy(r3.at[wid], rv)
    pltpu.sync_copy(t3.at[wid], tv)

    copies = []
    for k in range(NCHUNK):
        sl = pl.ds(k * CHUNK, CHUNK)
        copies.append(pltpu.async_copy(eEP.at[hv.at[k]], hp.at[sl], sem))
        copies.append(pltpu.async_copy(eE.at[hv.at[k]], he.at[sl], sem))
        copies.append(pltpu.async_copy(eEP.at[tv.at[k]], tp.at[sl], sem))
        copies.append(pltpu.async_copy(eE.at[tv.at[k]], te.at[sl], sem))
        copies.append(pltpu.async_copy(rEP.at[rv.at[k]], rp.at[sl], sem))
        copies.append(pltpu.async_copy(rE.at[rv.at[k]], reb.at[sl], sem))
    for c in copies:
        c.wait()

    h0 = pl.ds(0, 16)
    h1 = pl.ds(16, 16)

    def one_row(i):
        hp0, hp1 = hp[i, h0], hp[i, h1]
        he0, he1 = he[i, h0], he[i, h1]
        tp0, tp1 = tp[i, h0], tp[i, h1]
        te0, te1 = te[i, h0], te[i, h1]
        rp0, rp1 = rp[i, h0], rp[i, h1]
        re0, re1 = reb[i, h0], reb[i, h1]
        s_hp = jnp.sum(hp0 * hp0 + hp1 * hp1)
        s_he = jnp.sum(he0 * he0 + he1 * he1)
        d_h = jnp.sum(hp0 * he0 + hp1 * he1)
        s_tp = jnp.sum(tp0 * tp0 + tp1 * tp1)
        s_te = jnp.sum(te0 * te0 + te1 * te1)
        d_t = jnp.sum(tp0 * te0 + tp1 * te1)
        s_rp = jnp.sum(rp0 * rp0 + rp1 * rp1)
        s_re = jnp.sum(re0 * re0 + re1 * re1)
        one = jnp.float32(1.0)
        c_hp = jnp.minimum(one, _rsqrt(s_hp))
        c_he = jnp.minimum(one, _rsqrt(s_he))
        c_tp = jnp.minimum(one, _rsqrt(s_tp))
        c_te = jnp.minimum(one, _rsqrt(s_te))
        c_rp = jnp.minimum(one, _rsqrt(s_rp))
        c_re = jnp.minimum(one, _rsqrt(s_re))
        f_h = c_rp * c_hp * c_he * d_h
        f_t = c_rp * c_tp * c_te * d_t
        hp[i, h0] = f_h * rp0 + c_he * he0
        hp[i, h1] = f_h * rp1 + c_he * he1
        tp[i, h0] = f_t * rp0 + c_te * te0
        tp[i, h1] = f_t * rp1 + c_te * te1
        reb[i, h0] = c_re * re0
        reb[i, h1] = c_re * re1

    UNROLL = 4

    def block(b, carry):
        for u in range(UNROLL):
            one_row(b * UNROLL + u)
        return carry

    lax.fori_loop(0, RPW // UNROLL, block, 0)

    out_sl = pl.ds(wid * RPW, RPW)
    pltpu.sync_copy(hp, hout.at[out_sl])
    pltpu.sync_copy(reb, reout.at[out_sl])
    pltpu.sync_copy(tp, tout.at[out_sl])


def kernel(h, r, t, entityEmb, relationEmb, entityEmbP, relationEmbP):
    h3 = h.astype(jnp.int32).reshape(NW, NCHUNK, CHUNK)
    r3 = r.astype(jnp.int32).reshape(NW, NCHUNK, CHUNK)
    t3 = t.astype(jnp.int32).reshape(NW, NCHUNK, CHUNK)
    eE = _tc_pack(entityEmb.T)
    eEP = _tc_pack(entityEmbP.T)
    hout, reb, tout = _transd_sc(h3, r3, t3, eE, relationEmb,
                                 eEP, relationEmbP)
    return (hout, reb, tout)


# speedup vs baseline: 1.0015x; 1.0015x over previous
import functools

import jax
import jax.numpy as jnp
from jax import lax
from jax.experimental import pallas as pl
from jax.experimental.pallas import tpu as pltpu
from jax.experimental.pallas import tpu_sc as plsc

B = 16384
D = 32
ENT = 1000000
NC = 2
NS = 16
NW = NC * NS
RPW = B // NW
NCHUNK = 4
CHUNK = RPW // NCHUNK

TCB = 16000
TCT = 8000
TCG = 63


def _tc_body(x_hbm, o_hbm, vin, vout, vtin, vtout, sin, sout, stail):
    g = pl.program_id(0)
    par = g % 2

    def in_copy(gi, p):
        return pltpu.make_async_copy(
            x_hbm.at[:, pl.ds(gi * TCB, TCB)], vin.at[p], sin.at[p])

    def out_copy(gi, p):
        return pltpu.make_async_copy(
            vout.at[p], o_hbm.at[pl.ds(gi * TCB, TCB), :], sout.at[p])

    tail_in = pltpu.make_async_copy(
        x_hbm.at[:, pl.ds((TCG - 1) * TCB, TCT)], vtin, stail)
    tail_out = pltpu.make_async_copy(
        vtout, o_hbm.at[pl.ds((TCG - 1) * TCB, TCT), :], stail)

    @pl.when(g == 0)
    def _():
        in_copy(0, 0).start()
        in_copy(1, 1).start()

    @pl.when(g == TCG - 2)
    def _():
        tail_in.start()

    @pl.when(jnp.logical_and(g + 1 < TCG - 1, g > 0))
    def _():
        in_copy(g + 1, (g + 1) % 2).start()

    @pl.when(g >= 2)
    def _():
        out_copy(g - 2, par).wait()

    ident = jnp.eye(D, dtype=jnp.float32)

    def _mxu_t(x):
        return lax.dot_general(x, ident, (((0,), (0,)), ((), ())),
                               preferred_element_type=jnp.float32)

    @pl.when(g < TCG - 1)
    def _():
        in_copy(g, par).wait()
        vout[par, :, :] = _mxu_t(vin[par])

    @pl.when(g == TCG - 1)
    def _():
        tail_in.wait()
        vtout[...] = _mxu_t(vtin[...])

    @pl.when(g < TCG - 1)
    def _():
        out_copy(g, par).start()

    @pl.when(g == TCG - 1)
    def _():
        tail_out.start()
        out_copy(g - 1, 1 - par).wait()
        tail_out.wait()


_tc_pack = pl.pallas_call(
    _tc_body,
    grid=(TCG,),
    compiler_params=pltpu.CompilerParams(fuse_transposed_lhs_in_matmul=True),
    in_specs=[pl.BlockSpec(memory_space=pltpu.MemorySpace.HBM)],
    out_specs=pl.BlockSpec(memory_space=pltpu.MemorySpace.HBM),
    out_shape=jax.ShapeDtypeStruct((ENT, D), jnp.float32),
    scratch_shapes=[
        pltpu.VMEM((2, D, TCB), jnp.float32),
        pltpu.VMEM((2, TCB, D), jnp.float32),
        pltpu.VMEM((D, TCT), jnp.float32),
        pltpu.VMEM((TCT, D), jnp.float32),
        pltpu.SemaphoreType.DMA((2,)),
        pltpu.SemaphoreType.DMA((2,)),
        pltpu.SemaphoreType.DMA,
    ],
)


def _rsqrt(x):
    i = lax.bitcast_convert_type(x, jnp.int32)
    y = lax.bitcast_convert_type(jnp.int32(0x5F3759DF) - (i >> 1),
                                 jnp.float32)
    for _ in range(3):
        y = y * (1.5 - 0.5 * x * y * y)
    return y


@functools.partial(
    pl.kernel,
    mesh=plsc.VectorSubcoreMesh(core_axis_name="c", subcore_axis_name="s"),
    compiler_params=pltpu.CompilerParams(
        needs_layout_passes=False, use_tc_tiling_on_sc=False),
    out_type=(
        jax.ShapeDtypeStruct((B, D), jnp.float32),
        jax.ShapeDtypeStruct((B, D), jnp.float32),
        jax.ShapeDtypeStruct((B, D), jnp.float32),
    ),
    scratch_types=[
        pltpu.VMEM((NCHUNK, CHUNK), jnp.int32),
        pltpu.VMEM((NCHUNK, CHUNK), jnp.int32),
        pltpu.VMEM((NCHUNK, CHUNK), jnp.int32),
        pltpu.VMEM((RPW, D), jnp.float32),
        pltpu.VMEM((RPW, D), jnp.float32),
        pltpu.VMEM((RPW, D), jnp.float32),
        pltpu.VMEM((RPW, D), jnp.float32),
        pltpu.VMEM((RPW, D), jnp.float32),
        pltpu.VMEM((RPW, D), jnp.float32),
        pltpu.SemaphoreType.DMA,
    ],
)
def _transd_sc(h3, r3, t3, eE, rE, eEP, rEP, hout, reout, tout,
               hv, rv, tv, hp, he, tp, te, rp, reb, sem):
    wid = lax.axis_index("s") * NC + lax.axis_index("c")
    pltpu.sync_copy(h3.at[wid], hv)
    pltpu.sync_copy(r3.at[wid], rv)
    pltpu.sync_copy(t3.at[wid], tv)

    copies = []
    for k in range(NCHUNK):
        sl = pl.ds(k * CHUNK, CHUNK)
        copies.append(pltpu.async_copy(eEP.at[hv.at[k]], hp.at[sl], sem))
        copies.append(pltpu.async_copy(eE.at[hv.at[k]], he.at[sl], sem))
        copies.append(pltpu.async_copy(eEP.at[tv.at[k]], tp.at[sl], sem))
        copies.append(pltpu.async_copy(eE.at[tv.at[k]], te.at[sl], sem))
        copies.append(pltpu.async_copy(rEP.at[rv.at[k]], rp.at[sl], sem))
        copies.append(pltpu.async_copy(rE.at[rv.at[k]], reb.at[sl], sem))
    for c in copies:
        c.wait()

    h0 = pl.ds(0, 16)
    h1 = pl.ds(16, 16)

    def one_row(i):
        hp0, hp1 = hp[i, h0], hp[i, h1]
        he0, he1 = he[i, h0], he[i, h1]
        tp0, tp1 = tp[i, h0], tp[i, h1]
        te0, te1 = te[i, h0], te[i, h1]
        rp0, rp1 = rp[i, h0], rp[i, h1]
        re0, re1 = reb[i, h0], reb[i, h1]
        s_hp = jnp.sum(hp0 * hp0 + hp1 * hp1)
        s_he = jnp.sum(he0 * he0 + he1 * he1)
        d_h = jnp.sum(hp0 * he0 + hp1 * he1)
        s_tp = jnp.sum(tp0 * tp0 + tp1 * tp1)
        s_te = jnp.sum(te0 * te0 + te1 * te1)
        d_t = jnp.sum(tp0 * te0 + tp1 * te1)
        s_rp = jnp.sum(rp0 * rp0 + rp1 * rp1)
        s_re = jnp.sum(re0 * re0 + re1 * re1)
        one = jnp.float32(1.0)
        c_hp = jnp.minimum(one, _rsqrt(s_hp))
        c_he = jnp.minimum(one, _rsqrt(s_he))
        c_tp = jnp.minimum(one, _rsqrt(s_tp))
        c_te = jnp.minimum(one, _rsqrt(s_te))
        c_rp = jnp.minimum(one, _rsqrt(s_rp))
        c_re = jnp.minimum(one, _rsqrt(s_re))
        f_h = c_rp * c_hp * c_he * d_h
        f_t = c_rp * c_tp * c_te * d_t
        hp[i, h0] = f_h * rp0 + c_he * he0
        hp[i, h1] = f_h * rp1 + c_he * he1
        tp[i, h0] = f_t * rp0 + c_te * te0
        tp[i, h1] = f_t * rp1 + c_te * te1
        reb[i, h0] = c_re * re0
        reb[i, h1] = c_re * re1

    UNROLL = 4

    def block(b, carry):
        for u in range(UNROLL):
            one_row(b * UNROLL + u)
        return carry

    lax.fori_loop(0, RPW // UNROLL, block, 0)

    out_sl = pl.ds(wid * RPW, RPW)
    pltpu.sync_copy(hp, hout.at[out_sl])
    pltpu.sync_copy(reb, reout.at[out_sl])
    pltpu.sync_copy(tp, tout.at[out_sl])


def kernel(h, r, t, entityEmb, relationEmb, entityEmbP, relationEmbP):
    h3 = h.astype(jnp.int32).reshape(NW, NCHUNK, CHUNK)
    r3 = r.astype(jnp.int32).reshape(NW, NCHUNK, CHUNK)
    t3 = t.astype(jnp.int32).reshape(NW, NCHUNK, CHUNK)
    eE = _tc_pack(entityEmb.T)
    eEP = _tc_pack(entityEmbP.T)
    hout, reb, tout = _transd_sc(h3, r3, t3, eE, relationEmb,
                                 eEP, relationEmbP)
    return (hout, reb, tout)


# speedup vs baseline: 1.1030x; 1.1014x over previous
import functools

import jax
import jax.numpy as jnp
from jax import lax
from jax.experimental import pallas as pl
from jax.experimental.pallas import tpu as pltpu
from jax.experimental.pallas import tpu_sc as plsc

B = 16384
D = 32
ENT = 1000000
NC = 2
NS = 16
NW = NC * NS
RPW = B // NW
NCHUNK = 4
CHUNK = RPW // NCHUNK

TCB = 16000
TCT = 8000
TCG = 63


def _tc_body(x_hbm, o_hbm, vin, vout, vtin, vtout, sin, sout, stail):
    g = pl.program_id(0)
    par = g % 2

    def in_copies(gi, p):
        return [
            pltpu.make_async_copy(
                x_hbm.at[r, :, pl.ds(gi * TCB, TCB)], vin.at[p, r],
                sin.at[p])
            for r in range(4)
        ]

    def in_copy(gi, p):
        class _Group:
            def start(self):
                for c in in_copies(gi, p):
                    c.start()

            def wait(self):
                for c in in_copies(gi, p):
                    c.wait()

        return _Group()

    def out_copy(gi, p):
        return pltpu.make_async_copy(
            vout.at[p], o_hbm.at[pl.ds(gi * TCB, TCB), :], sout.at[p])

    tail_ins = [
        pltpu.make_async_copy(
            x_hbm.at[r, :, pl.ds((TCG - 1) * TCB, TCT)], vtin.at[r], stail)
        for r in range(4)
    ]

    class _TailIn:
        def start(self):
            for c in tail_ins:
                c.start()

        def wait(self):
            for c in tail_ins:
                c.wait()

    tail_in = _TailIn()
    tail_out = pltpu.make_async_copy(
        vtout, o_hbm.at[pl.ds((TCG - 1) * TCB, TCT), :], stail)

    @pl.when(g == 0)
    def _():
        in_copy(0, 0).start()
        in_copy(1, 1).start()

    @pl.when(g == TCG - 2)
    def _():
        tail_in.start()

    @pl.when(jnp.logical_and(g + 1 < TCG - 1, g > 0))
    def _():
        in_copy(g + 1, (g + 1) % 2).start()

    @pl.when(g >= 2)
    def _():
        out_copy(g - 2, par).wait()

    ident = jnp.eye(D, dtype=jnp.float32)

    def _mxu_t(x):
        return lax.dot_general(x, ident, (((0,), (0,)), ((), ())),
                               preferred_element_type=jnp.float32)

    @pl.when(g < TCG - 1)
    def _():
        in_copy(g, par).wait()
        vout[par, :, :] = _mxu_t(vin[par].reshape(D, TCB))

    @pl.when(g == TCG - 1)
    def _():
        tail_in.wait()
        vtout[...] = _mxu_t(vtin[...].reshape(D, TCT))

    @pl.when(g < TCG - 1)
    def _():
        out_copy(g, par).start()

    @pl.when(g == TCG - 1)
    def _():
        tail_out.start()
        out_copy(g - 1, 1 - par).wait()
        tail_out.wait()


_tc_pack = pl.pallas_call(
    _tc_body,
    grid=(TCG,),
    compiler_params=pltpu.CompilerParams(fuse_transposed_lhs_in_matmul=True),
    in_specs=[pl.BlockSpec(memory_space=pltpu.MemorySpace.HBM)],
    out_specs=pl.BlockSpec(memory_space=pltpu.MemorySpace.HBM),
    out_shape=jax.ShapeDtypeStruct((ENT, D), jnp.float32),
    scratch_shapes=[
        pltpu.VMEM((2, 4, 8, TCB), jnp.float32),
        pltpu.VMEM((2, TCB, D), jnp.float32),
        pltpu.VMEM((4, 8, TCT), jnp.float32),
        pltpu.VMEM((TCT, D), jnp.float32),
        pltpu.SemaphoreType.DMA((2,)),
        pltpu.SemaphoreType.DMA((2,)),
        pltpu.SemaphoreType.DMA,
    ],
)


def _rsqrt(x):
    i = lax.bitcast_convert_type(x, jnp.int32)
    y = lax.bitcast_convert_type(jnp.int32(0x5F3759DF) - (i >> 1),
                                 jnp.float32)
    for _ in range(3):
        y = y * (1.5 - 0.5 * x * y * y)
    return y


@functools.partial(
    pl.kernel,
    mesh=plsc.VectorSubcoreMesh(core_axis_name="c", subcore_axis_name="s"),
    compiler_params=pltpu.CompilerParams(
        needs_layout_passes=False, use_tc_tiling_on_sc=False),
    out_type=(
        jax.ShapeDtypeStruct((B, D), jnp.float32),
        jax.ShapeDtypeStruct((B, D), jnp.float32),
        jax.ShapeDtypeStruct((B, D), jnp.float32),
    ),
    scratch_types=[
        pltpu.VMEM((NCHUNK, CHUNK), jnp.int32),
        pltpu.VMEM((NCHUNK, CHUNK), jnp.int32),
        pltpu.VMEM((NCHUNK, CHUNK), jnp.int32),
        pltpu.VMEM((RPW, D), jnp.float32),
        pltpu.VMEM((RPW, D), jnp.float32),
        pltpu.VMEM((RPW, D), jnp.float32),
        pltpu.VMEM((RPW, D), jnp.float32),
        pltpu.VMEM((RPW, D), jnp.float32),
        pltpu.VMEM((RPW, D), jnp.float32),
        pltpu.SemaphoreType.DMA,
    ],
)
def _transd_sc(h3, r3, t3, eE, rE, eEP, rEP, hout, reout, tout,
               hv, rv, tv, hp, he, tp, te, rp, reb, sem):
    wid = lax.axis_index("s") * NC + lax.axis_index("c")
    pltpu.sync_copy(h3.at[wid], hv)
    pltpu.sync_copy(r3.at[wid], rv)
    pltpu.sync_copy(t3.at[wid], tv)

    copies = []
    for k in range(NCHUNK):
        sl = pl.ds(k * CHUNK, CHUNK)
        copies.append(pltpu.async_copy(eEP.at[hv.at[k]], hp.at[sl], sem))
        copies.append(pltpu.async_copy(eE.at[hv.at[k]], he.at[sl], sem))
        copies.append(pltpu.async_copy(eEP.at[tv.at[k]], tp.at[sl], sem))
        copies.append(pltpu.async_copy(eE.at[tv.at[k]], te.at[sl], sem))
        copies.append(pltpu.async_copy(rEP.at[rv.at[k]], rp.at[sl], sem))
        copies.append(pltpu.async_copy(rE.at[rv.at[k]], reb.at[sl], sem))
    for c in copies:
        c.wait()

    h0 = pl.ds(0, 16)
    h1 = pl.ds(16, 16)

    def one_row(i):
        hp0, hp1 = hp[i, h0], hp[i, h1]
        he0, he1 = he[i, h0], he[i, h1]
        tp0, tp1 = tp[i, h0], tp[i, h1]
        te0, te1 = te[i, h0], te[i, h1]
        rp0, rp1 = rp[i, h0], rp[i, h1]
        re0, re1 = reb[i, h0], reb[i, h1]
        s_hp = jnp.sum(hp0 * hp0 + hp1 * hp1)
        s_he = jnp.sum(he0 * he0 + he1 * he1)
        d_h = jnp.sum(hp0 * he0 + hp1 * he1)
        s_tp = jnp.sum(tp0 * tp0 + tp1 * tp1)
        s_te = jnp.sum(te0 * te0 + te1 * te1)
        d_t = jnp.sum(tp0 * te0 + tp1 * te1)
        s_rp = jnp.sum(rp0 * rp0 + rp1 * rp1)
        s_re = jnp.sum(re0 * re0 + re1 * re1)
        one = jnp.float32(1.0)
        c_hp = jnp.minimum(one, _rsqrt(s_hp))
        c_he = jnp.minimum(one, _rsqrt(s_he))
        c_tp = jnp.minimum(one, _rsqrt(s_tp))
        c_te = jnp.minimum(one, _rsqrt(s_te))
        c_rp = jnp.minimum(one, _rsqrt(s_rp))
        c_re = jnp.minimum(one, _rsqrt(s_re))
        f_h = c_rp * c_hp * c_he * d_h
        f_t = c_rp * c_tp * c_te * d_t
        hp[i, h0] = f_h * rp0 + c_he * he0
        hp[i, h1] = f_h * rp1 + c_he * he1
        tp[i, h0] = f_t * rp0 + c_te * te0
        tp[i, h1] = f_t * rp1 + c_te * te1
        reb[i, h0] = c_re * re0
        reb[i, h1] = c_re * re1

    UNROLL = 4

    def block(b, carry):
        for u in range(UNROLL):
            one_row(b * UNROLL + u)
        return carry

    lax.fori_loop(0, RPW // UNROLL, block, 0)

    out_sl = pl.ds(wid * RPW, RPW)
    pltpu.sync_copy(hp, hout.at[out_sl])
    pltpu.sync_copy(reb, reout.at[out_sl])
    pltpu.sync_copy(tp, tout.at[out_sl])


def kernel(h, r, t, entityEmb, relationEmb, entityEmbP, relationEmbP):
    h3 = h.astype(jnp.int32).reshape(NW, NCHUNK, CHUNK)
    r3 = r.astype(jnp.int32).reshape(NW, NCHUNK, CHUNK)
    t3 = t.astype(jnp.int32).reshape(NW, NCHUNK, CHUNK)
    eE = _tc_pack(entityEmb.T.reshape(4, 8, ENT))
    eEP = entityEmbP
    hout, reb, tout = _transd_sc(h3, r3, t3, eE, relationEmb,
                                 eEP, relationEmbP)
    return (hout, reb, tout)


# speedup vs baseline: 1.2160x; 1.1024x over previous
import functools

import jax
import jax.numpy as jnp
from jax import lax
from jax.experimental import pallas as pl
from jax.experimental.pallas import tpu as pltpu
from jax.experimental.pallas import tpu_sc as plsc

B = 16384
D = 32
ENT = 1000000
NC = 2
NS = 16
NW = NC * NS
RPW = B // NW
NCHUNK = 4
CHUNK = RPW // NCHUNK

def _rsqrt(x):
    i = lax.bitcast_convert_type(x, jnp.int32)
    y = lax.bitcast_convert_type(jnp.int32(0x5F3759DF) - (i >> 1),
                                 jnp.float32)
    for _ in range(3):
        y = y * (1.5 - 0.5 * x * y * y)
    return y


@functools.partial(
    pl.kernel,
    mesh=plsc.VectorSubcoreMesh(core_axis_name="c", subcore_axis_name="s"),
    compiler_params=pltpu.CompilerParams(
        needs_layout_passes=False, use_tc_tiling_on_sc=False),
    out_type=(
        jax.ShapeDtypeStruct((B, D), jnp.float32),
        jax.ShapeDtypeStruct((B, D), jnp.float32),
        jax.ShapeDtypeStruct((B, D), jnp.float32),
    ),
    scratch_types=[
        pltpu.VMEM((NCHUNK, CHUNK), jnp.int32),
        pltpu.VMEM((NCHUNK, CHUNK), jnp.int32),
        pltpu.VMEM((NCHUNK, CHUNK), jnp.int32),
        pltpu.VMEM((RPW, D), jnp.float32),
        pltpu.VMEM((RPW, D), jnp.float32),
        pltpu.VMEM((RPW, D), jnp.float32),
        pltpu.VMEM((RPW, D), jnp.float32),
        pltpu.VMEM((RPW, D), jnp.float32),
        pltpu.VMEM((RPW, D), jnp.float32),
        pltpu.SemaphoreType.DMA,
    ],
)
def _transd_sc(h3, r3, t3, eE, rE, eEP, rEP, hout, reout, tout,
               hv, rv, tv, hp, he, tp, te, rp, reb, sem):
    wid = lax.axis_index("s") * NC + lax.axis_index("c")
    pltpu.sync_copy(h3.at[wid], hv)
    pltpu.sync_copy(r3.at[wid], rv)
    pltpu.sync_copy(t3.at[wid], tv)

    copies = []
    for k in range(NCHUNK):
        sl = pl.ds(k * CHUNK, CHUNK)
        copies.append(pltpu.async_copy(eEP.at[hv.at[k]], hp.at[sl], sem))
        copies.append(pltpu.async_copy(eE.at[hv.at[k]], he.at[sl], sem))
        copies.append(pltpu.async_copy(eEP.at[tv.at[k]], tp.at[sl], sem))
        copies.append(pltpu.async_copy(eE.at[tv.at[k]], te.at[sl], sem))
        copies.append(pltpu.async_copy(rEP.at[rv.at[k]], rp.at[sl], sem))
        copies.append(pltpu.async_copy(rE.at[rv.at[k]], reb.at[sl], sem))
    for c in copies:
        c.wait()

    h0 = pl.ds(0, 16)
    h1 = pl.ds(16, 16)

    def one_row(i):
        hp0, hp1 = hp[i, h0], hp[i, h1]
        he0, he1 = he[i, h0], he[i, h1]
        tp0, tp1 = tp[i, h0], tp[i, h1]
        te0, te1 = te[i, h0], te[i, h1]
        rp0, rp1 = rp[i, h0], rp[i, h1]
        re0, re1 = reb[i, h0], reb[i, h1]
        s_hp = jnp.sum(hp0 * hp0 + hp1 * hp1)
        s_he = jnp.sum(he0 * he0 + he1 * he1)
        d_h = jnp.sum(hp0 * he0 + hp1 * he1)
        s_tp = jnp.sum(tp0 * tp0 + tp1 * tp1)
        s_te = jnp.sum(te0 * te0 + te1 * te1)
        d_t = jnp.sum(tp0 * te0 + tp1 * te1)
        s_rp = jnp.sum(rp0 * rp0 + rp1 * rp1)
        s_re = jnp.sum(re0 * re0 + re1 * re1)
        one = jnp.float32(1.0)
        c_hp = jnp.minimum(one, _rsqrt(s_hp))
        c_he = jnp.minimum(one, _rsqrt(s_he))
        c_tp = jnp.minimum(one, _rsqrt(s_tp))
        c_te = jnp.minimum(one, _rsqrt(s_te))
        c_rp = jnp.minimum(one, _rsqrt(s_rp))
        c_re = jnp.minimum(one, _rsqrt(s_re))
        f_h = c_rp * c_hp * c_he * d_h
        f_t = c_rp * c_tp * c_te * d_t
        hp[i, h0] = f_h * rp0 + c_he * he0
        hp[i, h1] = f_h * rp1 + c_he * he1
        tp[i, h0] = f_t * rp0 + c_te * te0
        tp[i, h1] = f_t * rp1 + c_te * te1
        reb[i, h0] = c_re * re0
        reb[i, h1] = c_re * re1

    UNROLL = 4

    def block(b, carry):
        for u in range(UNROLL):
            one_row(b * UNROLL + u)
        return carry

    lax.fori_loop(0, RPW // UNROLL, block, 0)

    out_sl = pl.ds(wid * RPW, RPW)
    pltpu.sync_copy(hp, hout.at[out_sl])
    pltpu.sync_copy(reb, reout.at[out_sl])
    pltpu.sync_copy(tp, tout.at[out_sl])


def kernel(h, r, t, entityEmb, relationEmb, entityEmbP, relationEmbP):
    h3 = h.astype(jnp.int32).reshape(NW, NCHUNK, CHUNK)
    r3 = r.astype(jnp.int32).reshape(NW, NCHUNK, CHUNK)
    t3 = t.astype(jnp.int32).reshape(NW, NCHUNK, CHUNK)
    eE = entityEmb
    eEP = entityEmbP
    hout, reb, tout = _transd_sc(h3, r3, t3, eE, relationEmb,
                                 eEP, relationEmbP)
    return (hout, reb, tout)
